# baseline (speedup 1.0000x reference)
import functools

import jax
import jax.numpy as jnp
from jax import lax
from jax.experimental import pallas as pl
from jax.experimental.pallas import tpu as pltpu
from jax.experimental.pallas import tpu_sc as plsc

_BATCH = 4096
_HIST = 200
_DIM = 64
_N = _BATCH * _HIST

_NC = 2
_NS = 16
_NW = _NC * _NS
_CH = 128
_NCHUNK = _N // (_NW * _CH)
_NBUF = 4
_NGRP = _NCHUNK // _NBUF


def _gather_body(ids_hbm, table_hbm, out_hbm, idx_v, rows, gsems, osems):
    wid = lax.axis_index("s") * _NC + lax.axis_index("c")

    pltpu.sync_copy(ids_hbm.at[wid], idx_v)

    def start_gather(slot, chunk):
        pltpu.async_copy(table_hbm.at[idx_v.at[chunk]], rows[slot], gsems[slot])

    def wait_gather(slot, chunk):
        pltpu.make_async_copy(
            table_hbm.at[idx_v.at[chunk]], rows[slot], gsems[slot]
        ).wait()

    def start_write(slot, chunk):
        pltpu.async_copy(rows[slot], out_hbm.at[wid, chunk], osems[slot])

    def wait_write(slot, chunk):
        pltpu.make_async_copy(
            rows[slot], out_hbm.at[wid, chunk], osems[slot]
        ).wait()

    for b in range(_NBUF):
        start_gather(b, b)

    def group(g, _):
        for b in range(_NBUF):
            c = g * _NBUF + b
            wait_gather(b, c)
            start_write(b, c)
            wait_write(b, c)
            start_gather(b, c + _NBUF)
        return _

    lax.fori_loop(0, _NGRP - 1, group, 0, unroll=False)

    g_last = _NGRP - 1
    for b in range(_NBUF):
        c = g_last * _NBUF + b
        wait_gather(b, c)
        start_write(b, c)
        wait_write(b, c)


@functools.partial(jax.jit, donate_argnums=())
def _run(ids3, table):
    mesh = plsc.VectorSubcoreMesh(core_axis_name="c", subcore_axis_name="s")
    f = functools.partial(
        pl.kernel,
        out_type=jax.ShapeDtypeStruct((_NW, _NCHUNK, _CH, _DIM), jnp.float32),
        mesh=mesh,
        scratch_types=[
            pltpu.VMEM((_NCHUNK, _CH), jnp.int32),
            [pltpu.VMEM((_CH, _DIM), jnp.float32) for _ in range(_NBUF)],
            [pltpu.SemaphoreType.DMA for _ in range(_NBUF)],
            [pltpu.SemaphoreType.DMA for _ in range(_NBUF)],
        ],
        compiler_params=pltpu.CompilerParams(use_tc_tiling_on_sc=False),
    )(_gather_body)
    return f(ids3, table)


def kernel(skill_ids, embeddings):
    ids3 = skill_ids.astype(jnp.int32).reshape(_NW, _NCHUNK, _CH)
    out = _run(ids3, embeddings)
    return out.reshape(_BATCH, _HIST, _DIM)


# speedup vs baseline: 1.2111x; 1.2111x over previous
import functools

import jax
import jax.numpy as jnp
from jax import lax
from jax.experimental import pallas as pl
from jax.experimental.pallas import tpu as pltpu
from jax.experimental.pallas import tpu_sc as plsc

_BATCH = 4096
_HIST = 200
_DIM = 64
_N = _BATCH * _HIST

_NC = 2
_NS = 16
_NW = _NC * _NS
_CH = 128
_NCHUNK = _N // (_NW * _CH)
_NBUF = 4
_NGRP = _NCHUNK // _NBUF


def _gather_body(ids_hbm, table_hbm, out_hbm, idx_v, rows, gsems, osems):
    wid = lax.axis_index("s") * _NC + lax.axis_index("c")

    pltpu.sync_copy(ids_hbm.at[wid], idx_v)

    def start_gather(slot, chunk):
        pltpu.async_copy(table_hbm.at[idx_v.at[chunk]], rows[slot], gsems[slot])

    def wait_gather(slot, chunk):
        pltpu.make_async_copy(
            table_hbm.at[idx_v.at[chunk]], rows[slot], gsems[slot]
        ).wait()

    def start_write(slot, chunk):
        base = (wid * _NCHUNK + chunk) * _CH
        pltpu.async_copy(
            rows[slot], out_hbm.at[pl.ds(base, _CH), pl.ds(0, _DIM)],
            osems[slot])
        pltpu.async_copy(
            rows[slot], out_hbm.at[pl.ds(base, _CH), pl.ds(_DIM, _DIM)],
            osems[slot])

    def wait_write(slot, chunk):
        base = (wid * _NCHUNK + chunk) * _CH
        pltpu.make_async_copy(
            rows[slot], out_hbm.at[pl.ds(base, _CH), pl.ds(0, _DIM)],
            osems[slot]).wait()
        pltpu.make_async_copy(
            rows[slot], out_hbm.at[pl.ds(base, _CH), pl.ds(_DIM, _DIM)],
            osems[slot]).wait()

    for b in range(_NBUF):
        start_gather(b, b)

    def group(g, _):
        for b in range(_NBUF):
            c = g * _NBUF + b
            wait_gather(b, c)
            start_write(b, c)
            wait_write(b, c)
            start_gather(b, c + _NBUF)
        return _

    lax.fori_loop(0, _NGRP - 1, group, 0, unroll=False)

    g_last = _NGRP - 1
    for b in range(_NBUF):
        c = g_last * _NBUF + b
        wait_gather(b, c)
        start_write(b, c)
        wait_write(b, c)


@functools.partial(jax.jit, donate_argnums=())
def _run(ids3, table):
    mesh = plsc.VectorSubcoreMesh(core_axis_name="c", subcore_axis_name="s")
    f = functools.partial(
        pl.kernel,
        out_type=jax.ShapeDtypeStruct((_N, 2 * _DIM), jnp.float32),
        mesh=mesh,
        scratch_types=[
            pltpu.VMEM((_NCHUNK, _CH), jnp.int32),
            [pltpu.VMEM((_CH, _DIM), jnp.float32) for _ in range(_NBUF)],
            [pltpu.SemaphoreType.DMA for _ in range(_NBUF)],
            [pltpu.SemaphoreType.DMA for _ in range(_NBUF)],
        ],
        compiler_params=pltpu.CompilerParams(use_tc_tiling_on_sc=False),
    )(_gather_body)
    return f(ids3, table)


def kernel(skill_ids, embeddings):
    ids3 = skill_ids.astype(jnp.int32).reshape(_NW, _NCHUNK, _CH)
    out = _run(ids3, embeddings)
    return out[:, :_DIM].reshape(_BATCH, _HIST, _DIM)


# speedup vs baseline: 1.2111x; 1.0000x over previous
import functools

import jax
import jax.numpy as jnp
from jax import lax
from jax.experimental import pallas as pl
from jax.experimental.pallas import tpu as pltpu
from jax.experimental.pallas import tpu_sc as plsc

_BATCH = 4096
_HIST = 200
_DIM = 64
_N = _BATCH * _HIST

_NC = 2
_NS = 16
_NW = _NC * _NS
_CH = 128
_NCHUNK = _N // (_NW * _CH)
_NBUF = 8
_NGRP = _NCHUNK // _NBUF


def _gather_body(ids_hbm, table_hbm, out_hbm, idx_v, rows, gsems, osems):
    wid = lax.axis_index("s") * _NC + lax.axis_index("c")

    pltpu.sync_copy(ids_hbm.at[wid], idx_v)

    def start_gather(slot, chunk):
        pltpu.async_copy(table_hbm.at[idx_v.at[chunk]], rows[slot], gsems[slot])

    def wait_gather(slot, chunk):
        pltpu.make_async_copy(
            table_hbm.at[idx_v.at[chunk]], rows[slot], gsems[slot]
        ).wait()

    def start_write(slot, chunk):
        base = (wid * _NCHUNK + chunk) * _CH
        pltpu.async_copy(
            rows[slot], out_hbm.at[pl.ds(base, _CH), pl.ds(0, _DIM)],
            osems[slot])
        pltpu.async_copy(
            rows[slot], out_hbm.at[pl.ds(base, _CH), pl.ds(_DIM, _DIM)],
            osems[slot])

    def wait_write(slot, chunk):
        base = (wid * _NCHUNK + chunk) * _CH
        pltpu.make_async_copy(
            rows[slot], out_hbm.at[pl.ds(base, _CH), pl.ds(0, _DIM)],
            osems[slot]).wait()
        pltpu.make_async_copy(
            rows[slot], out_hbm.at[pl.ds(base, _CH), pl.ds(_DIM, _DIM)],
            osems[slot]).wait()

    for b in range(_NBUF):
        start_gather(b, b)

    def group(g, _):
        for b in range(_NBUF):
            c = g * _NBUF + b
            wait_gather(b, c)
            start_write(b, c)
            wait_write(b, c)
            start_gather(b, c + _NBUF)
        return _

    lax.fori_loop(0, _NGRP - 1, group, 0, unroll=False)

    g_last = _NGRP - 1
    for b in range(_NBUF):
        c = g_last * _NBUF + b
        wait_gather(b, c)
        start_write(b, c)
        wait_write(b, c)


@functools.partial(jax.jit, donate_argnums=())
def _run(ids3, table):
    mesh = plsc.VectorSubcoreMesh(core_axis_name="c", subcore_axis_name="s")
    f = functools.partial(
        pl.kernel,
        out_type=jax.ShapeDtypeStruct((_N, 2 * _DIM), jnp.float32),
        mesh=mesh,
        scratch_types=[
            pltpu.VMEM((_NCHUNK, _CH), jnp.int32),
            [pltpu.VMEM((_CH, _DIM), jnp.float32) for _ in range(_NBUF)],
            [pltpu.SemaphoreType.DMA for _ in range(_NBUF)],
            [pltpu.SemaphoreType.DMA for _ in range(_NBUF)],
        ],
        compiler_params=pltpu.CompilerParams(use_tc_tiling_on_sc=False),
    )(_gather_body)
    return f(ids3, table)


def kernel(skill_ids, embeddings):
    ids3 = skill_ids.astype(jnp.int32).reshape(_NW, _NCHUNK, _CH)
    out = _run(ids3, embeddings)
    return out[:, :_DIM].reshape(_BATCH, _HIST, _DIM)


# speedup vs baseline: 1.3268x; 1.0955x over previous
import functools

import jax
import jax.numpy as jnp
from jax import lax
from jax.experimental import pallas as pl
from jax.experimental.pallas import tpu as pltpu
from jax.experimental.pallas import tpu_sc as plsc

_BATCH = 4096
_HIST = 200
_DIM = 64
_N = _BATCH * _HIST

_NC = 2
_NS = 16
_NW = _NC * _NS
_CH = 128
_NCHUNK = _N // (_NW * _CH)
_NBUF = 8
_NGRP = _NCHUNK // _NBUF


def _gather_body(ids_hbm, table_hbm, out_hbm, idx_v, rows, gsems, osems):
    wid = lax.axis_index("s") * _NC + lax.axis_index("c")

    pltpu.sync_copy(ids_hbm.at[wid], idx_v)

    def start_gather(slot, chunk):
        pltpu.async_copy(table_hbm.at[idx_v.at[chunk]], rows[slot], gsems[slot])

    def wait_gather(slot, chunk):
        pltpu.make_async_copy(
            table_hbm.at[idx_v.at[chunk]], rows[slot], gsems[slot]
        ).wait()

    def start_write(slot, chunk):
        base = (wid * _NCHUNK + chunk) * _CH
        pltpu.async_copy(
            rows[slot], out_hbm.at[pl.ds(base, _CH), pl.ds(0, _DIM)],
            osems[slot])

    def wait_write(slot, chunk):
        base = (wid * _NCHUNK + chunk) * _CH
        pltpu.make_async_copy(
            rows[slot], out_hbm.at[pl.ds(base, _CH), pl.ds(0, _DIM)],
            osems[slot]).wait()

    for b in range(_NBUF):
        start_gather(b, b)

    def group(g, _):
        for b in range(_NBUF):
            c = g * _NBUF + b
            wait_gather(b, c)
            start_write(b, c)
            wait_write(b, c)
            start_gather(b, c + _NBUF)
        return _

    lax.fori_loop(0, _NGRP - 1, group, 0, unroll=False)

    g_last = _NGRP - 1
    for b in range(_NBUF):
        c = g_last * _NBUF + b
        wait_gather(b, c)
        start_write(b, c)
        wait_write(b, c)


@functools.partial(jax.jit, donate_argnums=())
def _run(ids3, table):
    mesh = plsc.VectorSubcoreMesh(core_axis_name="c", subcore_axis_name="s")
    f = functools.partial(
        pl.kernel,
        out_type=jax.ShapeDtypeStruct((_N, 2 * _DIM), jnp.float32),
        mesh=mesh,
        scratch_types=[
            pltpu.VMEM((_NCHUNK, _CH), jnp.int32),
            [pltpu.VMEM((_CH, _DIM), jnp.float32) for _ in range(_NBUF)],
            [pltpu.SemaphoreType.DMA for _ in range(_NBUF)],
            [pltpu.SemaphoreType.DMA for _ in range(_NBUF)],
        ],
        compiler_params=pltpu.CompilerParams(use_tc_tiling_on_sc=False),
    )(_gather_body)
    return f(ids3, table)


def kernel(skill_ids, embeddings):
    ids3 = skill_ids.astype(jnp.int32).reshape(_NW, _NCHUNK, _CH)
    out = _run(ids3, embeddings)
    return out[:, :_DIM].reshape(_BATCH, _HIST, _DIM)
